# baseline (speedup 1.0000x reference)
import jax
import jax.numpy as jnp
from jax.experimental import pallas as pl
from jax.experimental.pallas import tpu as pltpu

_GRID = 50
_XB = 10000 // _GRID
_EB = 320000 // _GRID


def _copy_body(x_ref, ea_ref, xo_ref, eo_ref):
    xo_ref[...] = x_ref[...]
    eo_ref[...] = ea_ref[...]


def kernel(x, edge_index, edge_attr):
    x_out, ea_out = pl.pallas_call(
        _copy_body,
        grid=(_GRID,),
        out_shape=(
            jax.ShapeDtypeStruct((10000, 128), x.dtype),
            jax.ShapeDtypeStruct((320000, 16), edge_attr.dtype),
        ),
        in_specs=[
            pl.BlockSpec((_XB, 128), lambda i: (i, 0)),
            pl.BlockSpec((_EB, 16), lambda i: (i, 0)),
        ],
        out_specs=(
            pl.BlockSpec((_XB, 128), lambda i: (i, 0)),
            pl.BlockSpec((_EB, 16), lambda i: (i, 0)),
        ),
    )(x, edge_attr)
    return (x_out, ea_out)


# speedup vs baseline: 1.0107x; 1.0107x over previous
import jax
import jax.numpy as jnp
from jax.experimental import pallas as pl
from jax.experimental.pallas import tpu as pltpu

_XC = 8
_XR = 10000 // _XC
_EC = 32
_ER = 320000 // _EC
_ED = 8


def _copy_body(x_ref, ea_ref, xo_ref, eo_ref, xbuf, ebuf, sxi, sxo, sei, seo):
    def xin(i):
        sl = pl.ds(i * _XR, _XR)
        return pltpu.make_async_copy(x_ref.at[sl, :], xbuf.at[sl, :], sxi.at[i])

    def xout(i):
        sl = pl.ds(i * _XR, _XR)
        return pltpu.make_async_copy(xbuf.at[sl, :], xo_ref.at[sl, :], sxo.at[i])

    def ein(i):
        slot = pl.ds((i % _ED) * _ER, _ER)
        return pltpu.make_async_copy(
            ea_ref.at[pl.ds(i * _ER, _ER), :], ebuf.at[slot, :], sei.at[i % _ED])

    def eout(i):
        slot = pl.ds((i % _ED) * _ER, _ER)
        return pltpu.make_async_copy(
            ebuf.at[slot, :], eo_ref.at[pl.ds(i * _ER, _ER), :], seo.at[i % _ED])

    for i in range(_XC):
        xin(i).start()
    for i in range(_ED):
        ein(i).start()
    for i in range(_XC):
        xin(i).wait()
        xout(i).start()
    for i in range(_EC):
        ein(i).wait()
        eout(i).start()
        if i + _ED < _EC:
            eout(i).wait()
            ein(i + _ED).start()
    for i in range(_XC):
        xout(i).wait()
    for i in range(_EC - _ED, _EC):
        eout(i).wait()


def kernel(x, edge_index, edge_attr):
    x_out, ea_out = pl.pallas_call(
        _copy_body,
        out_shape=(
            jax.ShapeDtypeStruct((10000, 128), x.dtype),
            jax.ShapeDtypeStruct((320000, 16), edge_attr.dtype),
        ),
        in_specs=[
            pl.BlockSpec(memory_space=pl.ANY),
            pl.BlockSpec(memory_space=pl.ANY),
        ],
        out_specs=(
            pl.BlockSpec(memory_space=pl.ANY),
            pl.BlockSpec(memory_space=pl.ANY),
        ),
        scratch_shapes=[
            pltpu.MemorySpace.VMEM((10000, 128), jnp.float32),
            pltpu.MemorySpace.VMEM((_ED * _ER, 16), jnp.float32),
            pltpu.SemaphoreType.DMA((_XC,)),
            pltpu.SemaphoreType.DMA((_XC,)),
            pltpu.SemaphoreType.DMA((_ED,)),
            pltpu.SemaphoreType.DMA((_ED,)),
        ],
    )(x, edge_attr)
    return (x_out, ea_out)
